# Initial kernel scaffold; baseline (speedup 1.0000x reference)
import jax
import jax.numpy as jnp
from jax.experimental import pallas as pl


def kernel(x, edge_index, W1, b1, g1, be1, W2, b2, g2, be2):
    raise NotImplementedError("write your pallas kernel here")


# speedup vs baseline: 6.4809x; 6.4809x over previous
import functools

import jax
import jax.numpy as jnp
from jax import lax
from jax.experimental import pallas as pl
from jax.experimental.pallas import tpu as pltpu
from jax.experimental.pallas import tpu_sc as plsc

N_NODES = 100000
IN_DIM = 128
HID = 32
N_EDGES = 1600000


_ROW_BLK = 2000


def _mlp_body(x_ref, w1_ref, b1_ref, g1_ref, be1_ref, w2_ref, b2_ref,
              g2_ref, be2_ref, out_ref):
    h = jnp.dot(x_ref[...], w1_ref[...], preferred_element_type=jnp.float32)
    h = h + b1_ref[...]
    mu = jnp.mean(h, axis=-1, keepdims=True)
    var = jnp.mean((h - mu) ** 2, axis=-1, keepdims=True)
    h = (h - mu) / jnp.sqrt(var + 1e-5) * g1_ref[...] + be1_ref[...]
    h = jnp.maximum(h, 0.0)
    h = jnp.dot(h, w2_ref[...], preferred_element_type=jnp.float32)
    h = h + b2_ref[...]
    mu = jnp.mean(h, axis=-1, keepdims=True)
    var = jnp.mean((h - mu) ** 2, axis=-1, keepdims=True)
    h = (h - mu) / jnp.sqrt(var + 1e-5) * g2_ref[...] + be2_ref[...]
    out_ref[...] = jnp.maximum(h, 0.0)


def _mlp(x, W1, b1, g1, be1, W2, b2, g2, be2):
    n = x.shape[0]
    grid = (n // _ROW_BLK,)
    full = lambda shape: pl.BlockSpec(shape, lambda i: (0, 0))
    return pl.pallas_call(
        _mlp_body,
        grid=grid,
        in_specs=[
            pl.BlockSpec((_ROW_BLK, IN_DIM), lambda i: (i, 0)),
            full((IN_DIM, HID)),
            full((1, HID)), full((1, HID)), full((1, HID)),
            full((HID, HID)),
            full((1, HID)), full((1, HID)), full((1, HID)),
        ],
        out_specs=pl.BlockSpec((_ROW_BLK, HID), lambda i: (i, 0)),
        out_shape=jax.ShapeDtypeStruct((n, HID), jnp.float32),
    )(x, W1, b1.reshape(1, HID), g1.reshape(1, HID), be1.reshape(1, HID),
      W2, b2.reshape(1, HID), g2.reshape(1, HID), be2.reshape(1, HID))


_N_HALF = N_NODES // 2
_NS = 16
_ROWS_PT = (_N_HALF // _NS) // 8 * 8
_ROWS_REM = _N_HALF - _NS * _ROWS_PT
_DUMMY = _N_HALF
_K = 512
_R = _K // 128
_E_TILE = -(-N_EDGES // (_NS * _K)) * _K
_E_PAD = _E_TILE * _NS
_CHUNKS = _E_TILE // _K

@functools.cache
def _make_aggregate():
    mesh = plsc.VectorSubcoreMesh(core_axis_name="c", subcore_axis_name="s")
    return functools.partial(
        pl.kernel,
        mesh=mesh,
        out_type=jax.ShapeDtypeStruct((N_NODES, HID), jnp.float32),
        scratch_types=[
            pltpu.VMEM((_R, 128), jnp.int32),
            pltpu.VMEM((_R, 128), jnp.int32),
            pltpu.VMEM((_R, 128), jnp.int32),
            pltpu.VMEM((_K, HID), jnp.float32),
            pltpu.VMEM_SHARED((_N_HALF + 8, HID), jnp.float32),
            pltpu.SemaphoreType.DMA,
        ],
        compiler_params=pltpu.CompilerParams(
            use_tc_tiling_on_sc=False,
            internal_scratch_in_bytes=128 * 1024,
        ),
    )(_aggregate_body)


def _aggregate_body(h_hbm, row_hbm, col_hbm, out_hbm,
                    row_v, col_v, loc_v, rows_v, acc, sem):
    c = lax.axis_index("c")
    s = lax.axis_index("s")
    lo = c * _N_HALF
    pltpu.sync_copy(h_hbm.at[pl.ds(lo + s * _ROWS_PT, _ROWS_PT)],
                    acc.at[pl.ds(s * _ROWS_PT, _ROWS_PT)])

    @pl.when(s == 0)
    def _init_rem():
        pltpu.sync_copy(h_hbm.at[pl.ds(lo + _NS * _ROWS_PT, _ROWS_REM)],
                        acc.at[pl.ds(_NS * _ROWS_PT, _ROWS_REM)])

    plsc.subcore_barrier()

    row0 = s * (_E_TILE // 128)

    def chunk_body(i, carry):
        r0 = row0 + i * _R
        pltpu.sync_copy(row_hbm.at[pl.ds(r0, _R)], row_v)
        pltpu.sync_copy(col_hbm.at[pl.ds(r0, _R)], col_v)
        for jr in range(_R):
            for ji in range(8):
                r = row_v[jr, pl.ds(ji * 16, 16)]
                l = r - lo
                valid = (l >= 0) & (l < _N_HALF)
                loc_v[jr, pl.ds(ji * 16, 16)] = jnp.where(valid, l, _DUMMY)
        copies = [
            pltpu.async_copy(h_hbm.at[col_v.at[jr]],
                             rows_v.at[pl.ds(jr * 128, 128)], sem)
            for jr in range(_R)
        ]
        for cp in copies:
            cp.wait()
        for jr in range(_R):
            pltpu.sync_copy(rows_v.at[pl.ds(jr * 128, 128)],
                            acc.at[loc_v.at[jr]], add=True)
        return carry

    lax.fori_loop(0, _CHUNKS, chunk_body, 0)
    plsc.subcore_barrier()
    pltpu.sync_copy(acc.at[pl.ds(s * _ROWS_PT, _ROWS_PT)],
                    out_hbm.at[pl.ds(lo + s * _ROWS_PT, _ROWS_PT)])

    @pl.when(s == 0)
    def _out_rem():
        pltpu.sync_copy(acc.at[pl.ds(_NS * _ROWS_PT, _ROWS_REM)],
                        out_hbm.at[pl.ds(lo + _NS * _ROWS_PT, _ROWS_REM)])


def kernel(x, edge_index, W1, b1, g1, be1, W2, b2, g2, be2):
    h = _mlp(x, W1, b1, g1, be1, W2, b2, g2, be2)
    row = edge_index[0].astype(jnp.int32)
    col = edge_index[1].astype(jnp.int32)
    pad = _E_PAD - N_EDGES
    row = jnp.pad(row, (0, pad), constant_values=2 * N_NODES)
    col = jnp.pad(col, (0, pad), constant_values=0)
    row2d = row.reshape(_E_PAD // 128, 128)
    col2d = col.reshape(_E_PAD // 128, 128)
    return _make_aggregate()(h, row2d, col2d)
